# baseline (speedup 1.0000x reference)
import functools

import jax
import jax.numpy as jnp
from jax import lax
from jax.experimental import pallas as pl
from jax.experimental.pallas import tpu as pltpu
from jax.experimental.pallas import tpu_sc as plsc

NC = 2
NS = 16
NW = NC * NS
L = 16
CH = 128


@functools.partial(jax.jit, static_argnames=("batch", "k"))
def _mf_sc(user, item, user_f, item_f, user_b, item_b, mu1, *, batch, k):
    b_w = batch // NW
    nch = b_w // CH

    @functools.partial(
        pl.kernel,
        out_type=jax.ShapeDtypeStruct((batch,), jnp.float32),
        mesh=plsc.VectorSubcoreMesh(
            core_axis_name="c", subcore_axis_name="s",
            num_cores=NC, num_subcores=NS),
        scratch_types=[
            pltpu.VMEM((nch, CH), jnp.int32),
            pltpu.VMEM((nch, CH), jnp.int32),
            pltpu.VMEM((b_w, k), jnp.float32),
            pltpu.VMEM((b_w, k), jnp.float32),
            pltpu.VMEM((L * L,), jnp.float32),
            pltpu.VMEM((b_w,), jnp.float32),
            pltpu.VMEM((b_w,), jnp.float32),
            pltpu.VMEM((L,), jnp.float32),
            pltpu.VMEM((b_w,), jnp.float32),
            pltpu.SemaphoreType.DMA,
        ],
        compiler_params=pltpu.CompilerParams(
            needs_layout_passes=False, use_tc_tiling_on_sc=False),
    )
    def body(user_h, item_h, uf_h, if_h, ub_h, ib_h, mu_h, out_h,
             uidx_v, iidx_v, uf_v, if_v, tmp_v, ub_v, ib_v, mu_v, out_v, sem):
        wid = lax.axis_index("s") * NC + lax.axis_index("c")
        base = wid * b_w

        pltpu.sync_copy(mu_h, mu_v.at[pl.ds(0, 1)])
        for j in range(nch):
            pltpu.sync_copy(user_h.at[pl.ds(base + j * CH, CH)], uidx_v.at[j])
            pltpu.sync_copy(item_h.at[pl.ds(base + j * CH, CH)], iidx_v.at[j])

        copies = []
        for j in range(nch):
            sl = pl.ds(j * CH, CH)
            copies.append(pltpu.async_copy(
                uf_h.at[uidx_v.at[j]], uf_v.at[sl], sem))
            copies.append(pltpu.async_copy(
                if_h.at[iidx_v.at[j]], if_v.at[sl], sem))
            copies.append(pltpu.async_copy(
                ub_h.at[uidx_v.at[j]], ub_v.at[sl], sem))
            copies.append(pltpu.async_copy(
                ib_h.at[iidx_v.at[j]], ib_v.at[sl], sem))
        for c in copies:
            c.wait()

        mu_s = mu_v[pl.ds(0, L)][0]
        col = lax.iota(jnp.int32, L) * L
        nh = k // L

        def group(g, carry):
            r0 = g * L
            for rr in range(L):
                r = r0 + rr
                t = uf_v[r, pl.ds(0, L)] * if_v[r, pl.ds(0, L)]
                for h in range(1, nh):
                    t = t + uf_v[r, pl.ds(h * L, L)] * if_v[r, pl.ds(h * L, L)]
                plsc.store_scatter(tmp_v, [col + rr], t)
            acc = ub_v[pl.ds(r0, L)] + ib_v[pl.ds(r0, L)] + mu_s
            for j in range(L):
                acc = acc + tmp_v[pl.ds(j * L, L)]
            out_v[pl.ds(r0, L)] = acc
            return carry

        lax.fori_loop(0, b_w // L, group, 0)
        pltpu.sync_copy(out_v, out_h.at[pl.ds(base, b_w)])

    return body(user, item, user_f, item_f, user_b, item_b, mu1)


def kernel(user, item, user_f, item_f, user_b, item_b, mu):
    batch = user.shape[0]
    k = user_f.shape[1]
    mu1 = jnp.reshape(mu.astype(jnp.float32), (1,))
    ub = jnp.reshape(user_b, (user_b.shape[0],))
    ib = jnp.reshape(item_b, (item_b.shape[0],))
    return _mf_sc(user, item, user_f, item_f, ub, ib, mu1,
                  batch=batch, k=k)


# speedup vs baseline: 2.6318x; 2.6318x over previous
import functools

import jax
import jax.numpy as jnp
from jax import lax
from jax.experimental import pallas as pl
from jax.experimental.pallas import tpu as pltpu
from jax.experimental.pallas import tpu_sc as plsc

NC = 2
NS = 16
NW = NC * NS
L = 16
CH = 128
RING = 6


@functools.partial(jax.jit, static_argnames=("batch", "k"))
def _mf_sc(user, item, uf_t, if_t, user_b, item_b, mu1, *, batch, k):
    b_w = batch // NW
    nch = b_w // CH
    nblk = b_w // L
    nh = k // L

    @functools.partial(
        pl.kernel,
        out_type=jax.ShapeDtypeStruct((batch,), jnp.float32),
        mesh=plsc.VectorSubcoreMesh(
            core_axis_name="c", subcore_axis_name="s",
            num_cores=NC, num_subcores=NS),
        scratch_types=(
            [
                pltpu.VMEM((nch, CH), jnp.int32),
                pltpu.VMEM((nch, CH), jnp.int32),
                pltpu.VMEM((b_w * k,), jnp.float32),
                pltpu.VMEM((b_w * k,), jnp.float32),
                pltpu.VMEM((L * L,), jnp.float32),
                pltpu.VMEM((b_w,), jnp.float32),
                pltpu.VMEM((b_w,), jnp.float32),
                pltpu.VMEM((L,), jnp.float32),
                pltpu.VMEM((b_w,), jnp.float32),
            ]
            + [pltpu.VMEM((k, CH), jnp.float32) for _ in range(2 * RING)]
            + [pltpu.SemaphoreType.DMA for _ in range(2 * RING + 1)]
        ),
        compiler_params=pltpu.CompilerParams(
            needs_layout_passes=False, use_tc_tiling_on_sc=True),
    )
    def body(user_h, item_h, uf_h, if_h, ub_h, ib_h, mu_h, out_h,
             uidx_v, iidx_v, ufr_v, ifr_v, tmp_v, ub_v, ib_v, mu_v, out_v,
             *ws):
        uwin = ws[:RING]
        iwin = ws[RING:2 * RING]
        usem = ws[2 * RING:3 * RING]
        isem = ws[3 * RING:4 * RING]
        bsem = ws[4 * RING]
        wid = lax.axis_index("s") * NC + lax.axis_index("c")
        base = wid * b_w

        pltpu.sync_copy(mu_h, mu_v.at[pl.ds(0, 1)])
        for j in range(nch):
            pltpu.sync_copy(user_h.at[pl.ds(base + j * CH, CH)], uidx_v.at[j])
            pltpu.sync_copy(item_h.at[pl.ds(base + j * CH, CH)], iidx_v.at[j])

        bias_copies = []
        for j in range(nch):
            sl = pl.ds(j * CH, CH)
            bias_copies.append(pltpu.async_copy(
                ub_h.at[uidx_v.at[j]], ub_v.at[sl], bsem))
            bias_copies.append(pltpu.async_copy(
                ib_h.at[iidx_v.at[j]], ib_v.at[sl], bsem))

        kk0 = lax.iota(jnp.int32, L)

        def extract(win, sem, table_h, rows_v, ii, dest):
            pltpu.make_async_copy(
                table_h.at[:, pl.ds(0, CH)], win, sem).wait()
            iiv = jnp.zeros((L,), jnp.int32) + ii
            for h in range(nh):
                col = plsc.load_gather(win, [kk0 + h * L, iiv])
                rows_v[pl.ds(dest * k + h * L, L)] = col

        def blk(g, carry):
            cb = g // (CH // L)
            off = (g % (CH // L)) * L
            uvec = uidx_v[cb, pl.ds(off, L)]
            ivec = iidx_v[cb, pl.ds(off, L)]
            pend_u = [None] * RING
            pend_i = [None] * RING
            for l in range(L):
                su = uvec[l]
                si = ivec[l]
                s = l % RING
                if pend_u[s] is not None:
                    extract(uwin[s], usem[s], uf_h, ufr_v, *pend_u[s])
                    extract(iwin[s], isem[s], if_h, ifr_v, *pend_i[s])
                wu = pl.multiple_of(su & -CH, CH)
                wi = pl.multiple_of(si & -CH, CH)
                pltpu.async_copy(uf_h.at[:, pl.ds(wu, CH)], uwin[s], usem[s])
                pltpu.async_copy(if_h.at[:, pl.ds(wi, CH)], iwin[s], isem[s])
                dest = g * L + l
                pend_u[s] = (su & (CH - 1), dest)
                pend_i[s] = (si & (CH - 1), dest)
            for l in range(L - RING, L):
                s = l % RING
                extract(uwin[s], usem[s], uf_h, ufr_v, *pend_u[s])
                extract(iwin[s], isem[s], if_h, ifr_v, *pend_i[s])
            return carry

        lax.fori_loop(0, nblk, blk, 0)

        for c in bias_copies:
            c.wait()

        mu_s = mu_v[pl.ds(0, L)][0]
        col16 = kk0 * L

        def group(g, carry):
            r0 = g * L
            gb = r0 * k
            for rr in range(L):
                o = gb + rr * k
                t = ufr_v[pl.ds(o, L)] * ifr_v[pl.ds(o, L)]
                for h in range(1, nh):
                    t = t + ufr_v[pl.ds(o + h * L, L)] * ifr_v[pl.ds(o + h * L, L)]
                plsc.store_scatter(tmp_v, [col16 + rr], t)
            acc = ub_v[pl.ds(r0, L)] + ib_v[pl.ds(r0, L)] + mu_s
            for jj in range(L):
                acc = acc + tmp_v[pl.ds(jj * L, L)]
            out_v[pl.ds(r0, L)] = acc
            return carry

        lax.fori_loop(0, nblk, group, 0)
        pltpu.sync_copy(out_v, out_h.at[pl.ds(base, b_w)])

    return body(user, item, uf_t, if_t, user_b, item_b, mu1)


def kernel(user, item, user_f, item_f, user_b, item_b, mu):
    batch = user.shape[0]
    k = user_f.shape[1]
    mu1 = jnp.reshape(mu.astype(jnp.float32), (1,))
    ub = jnp.reshape(user_b, (user_b.shape[0],))
    ib = jnp.reshape(item_b, (item_b.shape[0],))
    return _mf_sc(user, item, user_f.T, item_f.T, ub, ib, mu1,
                  batch=batch, k=k)
